# baseline (speedup 1.0000x reference)
import functools

import jax
import jax.numpy as jnp
from jax import lax
from jax.experimental import pallas as pl
from jax.experimental.pallas import tpu as pltpu
from jax.experimental.pallas import tpu_sc as plsc

N = 10000
E = 320000
D_IN = 128
HID = 128
HEADS = 8
HP = 16
XP = HEADS * HID
NEG = 0.2

NC = 2
NS = 16
NW = NC * NS
EPW = E // NW
C1 = 80
NCH = EPW // C1
SUB = 16
NSUB = C1 // SUB
RPT = N // NS


def _ln(h, g, b):
    m = jnp.mean(h, axis=-1, keepdims=True)
    v = jnp.mean((h - m) * (h - m), axis=-1, keepdims=True)
    return (h - m) / jnp.sqrt(v + 1e-5) * g + b


def _gelu(h):
    return 0.5 * h * (1.0 + lax.erf(h * (2.0 ** -0.5)))


def _t0_body(x_ref, pw_ref, pb_ref, g_ref, b_ref, w_ref, as_ref, ad_ref,
             h_ref, xp_ref, es_ref, ed_ref):
    h = jnp.dot(x_ref[...], pw_ref[...], preferred_element_type=jnp.float32)
    h = _ln(h + pb_ref[...], g_ref[...], b_ref[...])
    h = _gelu(h)
    h_ref[...] = h
    xp = jnp.dot(h, w_ref[...], preferred_element_type=jnp.float32)
    xp_ref[...] = xp
    es_ref[...] = jnp.dot(xp, as_ref[...], preferred_element_type=jnp.float32)
    ed_ref[...] = jnp.dot(xp, ad_ref[...], preferred_element_type=jnp.float32)


def _t0_call(x, pw, pb, g, b, w, a_s, a_d):
    BN = 1000
    return pl.pallas_call(
        _t0_body,
        grid=(N // BN,),
        in_specs=[
            pl.BlockSpec((BN, D_IN), lambda i: (i, 0)),
            pl.BlockSpec((D_IN, HID), lambda i: (0, 0)),
            pl.BlockSpec((1, HID), lambda i: (0, 0)),
            pl.BlockSpec((1, HID), lambda i: (0, 0)),
            pl.BlockSpec((1, HID), lambda i: (0, 0)),
            pl.BlockSpec((HID, XP), lambda i: (0, 0)),
            pl.BlockSpec((XP, HP), lambda i: (0, 0)),
            pl.BlockSpec((XP, HP), lambda i: (0, 0)),
        ],
        out_specs=[
            pl.BlockSpec((BN, HID), lambda i: (i, 0)),
            pl.BlockSpec((BN, XP), lambda i: (i, 0)),
            pl.BlockSpec((BN, HP), lambda i: (i, 0)),
            pl.BlockSpec((BN, HP), lambda i: (i, 0)),
        ],
        out_shape=[
            jax.ShapeDtypeStruct((N, HID), jnp.float32),
            jax.ShapeDtypeStruct((N, XP), jnp.float32),
            jax.ShapeDtypeStruct((N, HP), jnp.float32),
            jax.ShapeDtypeStruct((N, HP), jnp.float32),
        ],
    )(x, pw, pb, g, b, w, a_s, a_d)


def _t2_body(sp_ref, r_ref):
    s = sp_ref[0] + sp_ref[1]
    col = lax.broadcasted_iota(jnp.int32, (N, HP), 1)
    r_ref[...] = jnp.where(col < HEADS, 0.125 / (s + 1e-16), 0.0)


def _t2_call(s_part):
    return pl.pallas_call(
        _t2_body,
        out_shape=jax.ShapeDtypeStruct((N, HP), jnp.float32),
    )(s_part)


def _t3_body(op_ref, pm_ref, bias_ref, hres_ref, g_ref, b_ref, w_ref,
             as_ref, ad_ref, h_ref, xp_ref, es_ref, ed_ref):
    agg = jnp.dot(op_ref[0] + op_ref[1], pm_ref[...],
                  preferred_element_type=jnp.float32)
    out = agg + bias_ref[...] + hres_ref[...]
    h = _ln(out, g_ref[...], b_ref[...])
    h = _gelu(h)
    h_ref[...] = h
    xp = jnp.dot(h, w_ref[...], preferred_element_type=jnp.float32)
    xp_ref[...] = xp
    es_ref[...] = jnp.dot(xp, as_ref[...], preferred_element_type=jnp.float32)
    ed_ref[...] = jnp.dot(xp, ad_ref[...], preferred_element_type=jnp.float32)


def _t3_call(op, pm, bias, hres, g, b, w, a_s, a_d):
    BN = 1000
    return pl.pallas_call(
        _t3_body,
        grid=(N // BN,),
        in_specs=[
            pl.BlockSpec((2, BN, HID), lambda i: (0, i, 0)),
            pl.BlockSpec((HID, HID), lambda i: (0, 0)),
            pl.BlockSpec((1, HID), lambda i: (0, 0)),
            pl.BlockSpec((BN, HID), lambda i: (i, 0)),
            pl.BlockSpec((1, HID), lambda i: (0, 0)),
            pl.BlockSpec((1, HID), lambda i: (0, 0)),
            pl.BlockSpec((HID, XP), lambda i: (0, 0)),
            pl.BlockSpec((XP, HP), lambda i: (0, 0)),
            pl.BlockSpec((XP, HP), lambda i: (0, 0)),
        ],
        out_specs=[
            pl.BlockSpec((BN, HID), lambda i: (i, 0)),
            pl.BlockSpec((BN, XP), lambda i: (i, 0)),
            pl.BlockSpec((BN, HP), lambda i: (i, 0)),
            pl.BlockSpec((BN, HP), lambda i: (i, 0)),
        ],
        out_shape=[
            jax.ShapeDtypeStruct((N, HID), jnp.float32),
            jax.ShapeDtypeStruct((N, XP), jnp.float32),
            jax.ShapeDtypeStruct((N, HP), jnp.float32),
            jax.ShapeDtypeStruct((N, HP), jnp.float32),
        ],
    )(op, pm, bias, hres, g, b, w, a_s, a_d)


def _ta_body(al8_ref, a_ref):
    a_ref[...] = jnp.sum(al8_ref[...], axis=1, keepdims=True)


def _ta_call(al8):
    BE = 20000
    return pl.pallas_call(
        _ta_body,
        grid=(E // BE,),
        in_specs=[pl.BlockSpec((BE, HP), lambda i: (i, 0))],
        out_specs=[pl.BlockSpec((BE, 1), lambda i: (i, 0))],
        out_shape=[jax.ShapeDtypeStruct((E, 1), jnp.float32)],
    )(al8)[0].reshape(E)


def _tf_body(op_ref, pm_ref, bias_ref, hres_ref, g_ref, b_ref, h_ref):
    agg = jnp.dot(op_ref[0] + op_ref[1], pm_ref[...],
                  preferred_element_type=jnp.float32)
    out = agg + bias_ref[...] + hres_ref[...]
    h_ref[...] = _ln(out, g_ref[...], b_ref[...])


def _tf_call(op, pm, bias, hres, g, b):
    BN = 1000
    return pl.pallas_call(
        _tf_body,
        grid=(N // BN,),
        in_specs=[
            pl.BlockSpec((2, BN, HID), lambda i: (0, i, 0)),
            pl.BlockSpec((HID, HID), lambda i: (0, 0)),
            pl.BlockSpec((1, HID), lambda i: (0, 0)),
            pl.BlockSpec((BN, HID), lambda i: (i, 0)),
            pl.BlockSpec((1, HID), lambda i: (0, 0)),
            pl.BlockSpec((1, HID), lambda i: (0, 0)),
        ],
        out_specs=[pl.BlockSpec((BN, HID), lambda i: (i, 0))],
        out_shape=[jax.ShapeDtypeStruct((N, HID), jnp.float32)],
    )(op, pm, bias, hres, g, b)[0]


def _s1_body(src_hbm, dst_hbm, es_hbm, ed_hbm,
             ex_hbm, sp_hbm,
             srcv_a, srcv_b, dstv_a, dstv_b, ga_a, ga_b, gb_a, gb_b,
             exv, zv, sacc, sem_sd, sem_g, sem_w):
    cid = lax.axis_index("c")
    sid = lax.axis_index("s")
    wid = sid * NC + cid

    def zrow(i, _):
        zv[i, :] = jnp.zeros((16,), jnp.float32)
        return 0
    lax.fori_loop(0, RPT, zrow, 0)
    pltpu.sync_copy(zv, sacc.at[pl.ds(sid * RPT, RPT)])
    plsc.subcore_barrier()

    base0 = wid * EPW

    pltpu.sync_copy(src_hbm.at[pl.ds(base0, C1)], srcv_a)
    pltpu.sync_copy(dst_hbm.at[pl.ds(base0, C1)], dstv_a)
    pltpu.async_copy(es_hbm.at[srcv_a], ga_a, sem_g)
    pltpu.async_copy(ed_hbm.at[dstv_a], gb_a, sem_g)

    def work(k, sv, dv, gav, gbv, svn, dvn, gan, gbn):
        b = base0 + k * C1
        more = k + 1 < NCH
        bn = b + C1

        @pl.when(more)
        def _():
            pltpu.async_copy(src_hbm.at[pl.ds(bn, C1)], svn, sem_sd)
            pltpu.async_copy(dst_hbm.at[pl.ds(bn, C1)], dvn, sem_sd)
        pltpu.make_async_copy(es_hbm.at[sv], gav, sem_g).wait()
        pltpu.make_async_copy(ed_hbm.at[dv], gbv, sem_g).wait()

        @pl.when(k >= 1)
        def _():
            pltpu.make_async_copy(
                exv, ex_hbm.at[pl.ds(base0, C1)], sem_w).wait()

        def erow(i, _):
            e = gav[i, :] + gbv[i, :]
            e = jnp.where(e >= 0.0, e, NEG * e)
            exv[i, :] = jnp.exp(e)
            return 0
        lax.fori_loop(0, C1, erow, 0)
        pltpu.async_copy(exv, ex_hbm.at[pl.ds(b, C1)], sem_w)
        pltpu.sync_copy(exv, sacc.at[dv], add=True)

        @pl.when(more)
        def _():
            pltpu.make_async_copy(src_hbm.at[pl.ds(bn, C1)], svn, sem_sd).wait()
            pltpu.make_async_copy(dst_hbm.at[pl.ds(bn, C1)], dvn, sem_sd).wait()
            pltpu.async_copy(es_hbm.at[svn], gan, sem_g)
            pltpu.async_copy(ed_hbm.at[dvn], gbn, sem_g)

    def chunk(k, _):
        q = k % 2

        @pl.when(q == 0)
        def _():
            work(k, srcv_a, dstv_a, ga_a, gb_a,
                 srcv_b, dstv_b, ga_b, gb_b)

        @pl.when(q == 1)
        def _():
            work(k, srcv_b, dstv_b, ga_b, gb_b,
                 srcv_a, dstv_a, ga_a, gb_a)
        return 0
    lax.fori_loop(0, NCH, chunk, 0)
    pltpu.make_async_copy(exv, ex_hbm.at[pl.ds(base0, C1)], sem_w).wait()
    plsc.subcore_barrier()

    @pl.when(sid == 0)
    def _():
        pltpu.sync_copy(sacc, sp_hbm.at[cid])


def _s1_call(src, dst, es, ed):
    k = pl.kernel(
        _s1_body,
        out_type=[
            jax.ShapeDtypeStruct((E, HP), jnp.float32),
            jax.ShapeDtypeStruct((NC, N, HP), jnp.float32),
        ],
        mesh=plsc.VectorSubcoreMesh(
            core_axis_name="c", subcore_axis_name="s",
            num_cores=NC, num_subcores=NS),
        compiler_params=pltpu.CompilerParams(use_tc_tiling_on_sc=False),
        scratch_types=[
            pltpu.VMEM((C1,), jnp.int32),
            pltpu.VMEM((C1,), jnp.int32),
            pltpu.VMEM((C1,), jnp.int32),
            pltpu.VMEM((C1,), jnp.int32),
            pltpu.VMEM((C1, HP), jnp.float32),
            pltpu.VMEM((C1, HP), jnp.float32),
            pltpu.VMEM((C1, HP), jnp.float32),
            pltpu.VMEM((C1, HP), jnp.float32),
            pltpu.VMEM((C1, HP), jnp.float32),
            pltpu.VMEM((RPT, HP), jnp.float32),
            pltpu.VMEM_SHARED((N, HP), jnp.float32),
            pltpu.SemaphoreType.DMA,
            pltpu.SemaphoreType.DMA,
            pltpu.SemaphoreType.DMA,
        ],
    )
    return k(src, dst, es, ed)


def _s2_body(src_hbm, dst_hbm, ex_hbm, r8_hbm, xp_hbm, outs, scratch):
    op_hbm = outs[0]
    al_hbm = outs[1] if len(outs) > 1 else None
    (srcv_a, srcv_b, dstv_a, dstv_b, exv, rv, av, xg0, xg1, msg0, msg1,
     zv, oacc, gs0, gs1, ss0, ss1, sem_e, sem_r, sem_sd) = scratch
    xg = [xg0, xg1]
    msg = [msg0, msg1]
    gsem = [gs0, gs1]
    ssem = [ss0, ss1]
    cid = lax.axis_index("c")
    sid = lax.axis_index("s")
    wid = sid * NC + cid

    def zrow(i, _):
        for j in range(HID // 16):
            zv[i, pl.ds(j * 16, 16)] = jnp.zeros((16,), jnp.float32)
        return 0
    lax.fori_loop(0, 25, zrow, 0)

    def zcp(r, _):
        pltpu.sync_copy(zv, oacc.at[pl.ds(sid * RPT + r * 25, 25)])
        return 0
    lax.fori_loop(0, RPT // 25, zcp, 0)
    plsc.subcore_barrier()

    base0 = wid * EPW

    pltpu.sync_copy(src_hbm.at[pl.ds(base0, C1)], srcv_a)
    pltpu.sync_copy(dst_hbm.at[pl.ds(base0, C1)], dstv_a)
    pltpu.async_copy(ex_hbm.at[pl.ds(base0, C1)], exv, sem_e)
    pltpu.async_copy(r8_hbm.at[dstv_a], rv, sem_r)

    def subblock(k, sv, dv, svn, dvn):
        more = k + 1 < NCH
        bn = base0 + (k + 1) * C1
        gd = [None, None]
        sd = [None, None]
        gd[0] = pltpu.async_copy(
            xp_hbm.at[sv.at[pl.ds(0, SUB)]], xg[0], gsem[0])
        for s in range(NSUB):
            p = s % 2
            if s + 1 < NSUB:
                gd[1 - p] = pltpu.async_copy(
                    xp_hbm.at[sv.at[pl.ds((s + 1) * SUB, SUB)]],
                    xg[1 - p], gsem[1 - p])
            if s == 0:
                @pl.when(more)
                def _():
                    pltpu.async_copy(src_hbm.at[pl.ds(bn, C1)], svn, sem_sd)
                    pltpu.async_copy(dst_hbm.at[pl.ds(bn, C1)], dvn, sem_sd)
            if s == 3:
                @pl.when(more)
                def _():
                    pltpu.make_async_copy(
                        src_hbm.at[pl.ds(bn, C1)], svn, sem_sd).wait()
                    pltpu.make_async_copy(
                        dst_hbm.at[pl.ds(bn, C1)], dvn, sem_sd).wait()
                    pltpu.async_copy(ex_hbm.at[pl.ds(bn, C1)], exv, sem_e)
                    pltpu.async_copy(r8_hbm.at[dvn], rv, sem_r)
            gd[p].wait()
            if s >= 2:
                sd[p].wait()
            xgp = xg[p]
            msgp = msg[p]

            def edge(e2, _):
                arow = av[s * SUB + e2, :]
                for j in range(HID // 16):
                    acc = arow[0] * xgp[e2, pl.ds(j * 16, 16)]
                    for h in range(1, HEADS):
                        acc = acc + arow[h] * xgp[e2, pl.ds(h * HID + j * 16, 16)]
                    msgp[e2, pl.ds(j * 16, 16)] = acc
                return 0
            lax.fori_loop(0, SUB, edge, 0)
            sd[p] = pltpu.async_copy(
                msgp, oacc.at[dv[pl.ds(s * SUB, SUB)]], ssem[p], add=True)
        sd[(NSUB - 2) % 2].wait()
        sd[(NSUB - 1) % 2].wait()

    def chunk(k, _):
        q = k % 2
        b = base0 + k * C1
        pltpu.make_async_copy(ex_hbm.at[pl.ds(b, C1)], exv, sem_e).wait()

        @pl.when(q == 0)
        def _():
            pltpu.make_async_copy(r8_hbm.at[dstv_a], rv, sem_r).wait()

        @pl.when(q == 1)
        def _():
            pltpu.make_async_copy(r8_hbm.at[dstv_b], rv, sem_r).wait()

        def arow(i, _):
            av[i, :] = exv[i, :] * rv[i, :]
            return 0
        lax.fori_loop(0, C1, arow, 0)
        if al_hbm is not None:
            pltpu.sync_copy(av, al_hbm.at[pl.ds(b, C1)])

        @pl.when(q == 0)
        def _():
            subblock(k, srcv_a, dstv_a, srcv_b, dstv_b)

        @pl.when(q == 1)
        def _():
            subblock(k, srcv_b, dstv_b, srcv_a, dstv_a)
        return 0
    lax.fori_loop(0, NCH, chunk, 0)
    plsc.subcore_barrier()

    @pl.when(sid == 0)
    def _():
        pltpu.sync_copy(oacc, op_hbm.at[cid])


def _s2_call(src, dst, ex, r8, xp, want_alpha):
    out_type = [jax.ShapeDtypeStruct((NC, N, HID), jnp.float32)]
    if want_alpha:
        out_type.append(jax.ShapeDtypeStruct((E, HP), jnp.float32))

    def body(*refs):
        ins = refs[:5]
        outs = refs[5:5 + len(out_type)]
        scratch = refs[5 + len(out_type):]
        _s2_body(*ins, outs, scratch)

    k = pl.kernel(
        body,
        out_type=out_type,
        mesh=plsc.VectorSubcoreMesh(
            core_axis_name="c", subcore_axis_name="s",
            num_cores=NC, num_subcores=NS),
        compiler_params=pltpu.CompilerParams(use_tc_tiling_on_sc=False),
        scratch_types=[
            pltpu.VMEM((C1,), jnp.int32),
            pltpu.VMEM((C1,), jnp.int32),
            pltpu.VMEM((C1,), jnp.int32),
            pltpu.VMEM((C1,), jnp.int32),
            pltpu.VMEM((C1, HP), jnp.float32),
            pltpu.VMEM((C1, HP), jnp.float32),
            pltpu.VMEM((C1, HP), jnp.float32),
            pltpu.VMEM((SUB, XP), jnp.float32),
            pltpu.VMEM((SUB, XP), jnp.float32),
            pltpu.VMEM((SUB, HID), jnp.float32),
            pltpu.VMEM((SUB, HID), jnp.float32),
            pltpu.VMEM((25, HID), jnp.float32),
            pltpu.VMEM_SHARED((N, HID), jnp.float32),
            pltpu.SemaphoreType.DMA,
            pltpu.SemaphoreType.DMA,
            pltpu.SemaphoreType.DMA,
            pltpu.SemaphoreType.DMA,
            pltpu.SemaphoreType.DMA,
            pltpu.SemaphoreType.DMA,
            pltpu.SemaphoreType.DMA,
        ],
    )
    return k(src, dst, ex, r8, xp)


def _unperm_mat():
    return jnp.eye(HID, dtype=jnp.float32)


def _att_mats(a_src, a_dst):
    eye = jnp.eye(HEADS, dtype=jnp.float32)
    asm = jnp.einsum('hc,hk->hck', a_src, eye).reshape(XP, HEADS)
    adm = jnp.einsum('hc,hk->hck', a_dst, eye).reshape(XP, HEADS)
    pad = jnp.zeros((XP, HP - HEADS), jnp.float32)
    return (jnp.concatenate([asm, pad], axis=1),
            jnp.concatenate([adm, pad], axis=1))


def kernel(x, edge_index, proj_W, proj_b, ln_in_g, ln_in_b,
           W0, att_src0, att_dst0, bias0, ln0_g, ln0_b,
           W1, att_src1, att_dst1, bias1, ln1_g, ln1_b,
           W2, att_src2, att_dst2, bias2, ln2_g, ln2_b):
    src = edge_index[0]
    dst = edge_index[1]
    Ws = [W0, W1, W2]
    atts = [(att_src0, att_dst0), (att_src1, att_dst1), (att_src2, att_dst2)]
    biases = [bias0, bias1, bias2]
    lns = [(ln0_g, ln0_b), (ln1_g, ln1_b), (ln2_g, ln2_b)]

    pm = _unperm_mat()
    a_s0, a_d0 = _att_mats(*atts[0])
    h, xp, es, ed = _t0_call(
        x, proj_W, proj_b.reshape(1, HID),
        ln_in_g.reshape(1, HID), ln_in_b.reshape(1, HID), W0, a_s0, a_d0)

    alpha = None
    for i in range(3):
        ex, s_part = _s1_call(src, dst, es, ed)
        r8 = _t2_call(s_part)
        if i < 2:
            op = _s2_call(src, dst, ex, r8, xp, False)[0]
            a_s, a_d = _att_mats(*atts[i + 1])
            h, xp, es, ed = _t3_call(
                op, pm, biases[i].reshape(1, HID), h,
                lns[i][0].reshape(1, HID), lns[i][1].reshape(1, HID),
                Ws[i + 1], a_s, a_d)
        else:
            op, al8 = _s2_call(src, dst, ex, r8, xp, True)
            alpha = _ta_call(al8)
            h = _tf_call(op, pm, biases[i].reshape(1, HID), h,
                         lns[i][0].reshape(1, HID), lns[i][1].reshape(1, HID))
    return h, alpha


# speedup vs baseline: 1.0021x; 1.0021x over previous
import functools

import jax
import jax.numpy as jnp
from jax import lax
from jax.experimental import pallas as pl
from jax.experimental.pallas import tpu as pltpu
from jax.experimental.pallas import tpu_sc as plsc

N = 10000
E = 320000
D_IN = 128
HID = 128
HEADS = 8
HP = 16
XP = HEADS * HID
NEG = 0.2

NC = 2
NS = 16
NW = NC * NS
EPW = E // NW
C1 = 80
NCH = EPW // C1
SUB = 16
NSUB = C1 // SUB
RPT = N // NS


def _ln(h, g, b):
    m = jnp.mean(h, axis=-1, keepdims=True)
    v = jnp.mean((h - m) * (h - m), axis=-1, keepdims=True)
    return (h - m) / jnp.sqrt(v + 1e-5) * g + b


def _gelu(h):
    return 0.5 * h * (1.0 + lax.erf(h * (2.0 ** -0.5)))


def _t0_body(x_ref, pw_ref, pb_ref, g_ref, b_ref, w_ref, as_ref, ad_ref,
             h_ref, xp_ref, es_ref, ed_ref):
    h = jnp.dot(x_ref[...], pw_ref[...], preferred_element_type=jnp.float32)
    h = _ln(h + pb_ref[...], g_ref[...], b_ref[...])
    h = _gelu(h)
    h_ref[...] = h
    xp = jnp.dot(h, w_ref[...], preferred_element_type=jnp.float32)
    xp_ref[...] = xp
    es_ref[...] = jnp.dot(xp, as_ref[...], preferred_element_type=jnp.float32)
    ed_ref[...] = jnp.dot(xp, ad_ref[...], preferred_element_type=jnp.float32)


def _t0_call(x, pw, pb, g, b, w, a_s, a_d):
    BN = 2000
    return pl.pallas_call(
        _t0_body,
        grid=(N // BN,),
        in_specs=[
            pl.BlockSpec((BN, D_IN), lambda i: (i, 0)),
            pl.BlockSpec((D_IN, HID), lambda i: (0, 0)),
            pl.BlockSpec((1, HID), lambda i: (0, 0)),
            pl.BlockSpec((1, HID), lambda i: (0, 0)),
            pl.BlockSpec((1, HID), lambda i: (0, 0)),
            pl.BlockSpec((HID, XP), lambda i: (0, 0)),
            pl.BlockSpec((XP, HP), lambda i: (0, 0)),
            pl.BlockSpec((XP, HP), lambda i: (0, 0)),
        ],
        out_specs=[
            pl.BlockSpec((BN, HID), lambda i: (i, 0)),
            pl.BlockSpec((BN, XP), lambda i: (i, 0)),
            pl.BlockSpec((BN, HP), lambda i: (i, 0)),
            pl.BlockSpec((BN, HP), lambda i: (i, 0)),
        ],
        out_shape=[
            jax.ShapeDtypeStruct((N, HID), jnp.float32),
            jax.ShapeDtypeStruct((N, XP), jnp.float32),
            jax.ShapeDtypeStruct((N, HP), jnp.float32),
            jax.ShapeDtypeStruct((N, HP), jnp.float32),
        ],
    )(x, pw, pb, g, b, w, a_s, a_d)


def _t2_body(sp_ref, r_ref):
    s = sp_ref[0] + sp_ref[1]
    col = lax.broadcasted_iota(jnp.int32, (N, HP), 1)
    r_ref[...] = jnp.where(col < HEADS, 0.125 / (s + 1e-16), 0.0)


def _t2_call(s_part):
    return pl.pallas_call(
        _t2_body,
        out_shape=jax.ShapeDtypeStruct((N, HP), jnp.float32),
    )(s_part)


def _t3_body(op_ref, pm_ref, bias_ref, hres_ref, g_ref, b_ref, w_ref,
             as_ref, ad_ref, h_ref, xp_ref, es_ref, ed_ref):
    agg = jnp.dot(op_ref[0] + op_ref[1], pm_ref[...],
                  preferred_element_type=jnp.float32)
    out = agg + bias_ref[...] + hres_ref[...]
    h = _ln(out, g_ref[...], b_ref[...])
    h = _gelu(h)
    h_ref[...] = h
    xp = jnp.dot(h, w_ref[...], preferred_element_type=jnp.float32)
    xp_ref[...] = xp
    es_ref[...] = jnp.dot(xp, as_ref[...], preferred_element_type=jnp.float32)
    ed_ref[...] = jnp.dot(xp, ad_ref[...], preferred_element_type=jnp.float32)


def _t3_call(op, pm, bias, hres, g, b, w, a_s, a_d):
    BN = 2000
    return pl.pallas_call(
        _t3_body,
        grid=(N // BN,),
        in_specs=[
            pl.BlockSpec((2, BN, HID), lambda i: (0, i, 0)),
            pl.BlockSpec((HID, HID), lambda i: (0, 0)),
            pl.BlockSpec((1, HID), lambda i: (0, 0)),
            pl.BlockSpec((BN, HID), lambda i: (i, 0)),
            pl.BlockSpec((1, HID), lambda i: (0, 0)),
            pl.BlockSpec((1, HID), lambda i: (0, 0)),
            pl.BlockSpec((HID, XP), lambda i: (0, 0)),
            pl.BlockSpec((XP, HP), lambda i: (0, 0)),
            pl.BlockSpec((XP, HP), lambda i: (0, 0)),
        ],
        out_specs=[
            pl.BlockSpec((BN, HID), lambda i: (i, 0)),
            pl.BlockSpec((BN, XP), lambda i: (i, 0)),
            pl.BlockSpec((BN, HP), lambda i: (i, 0)),
            pl.BlockSpec((BN, HP), lambda i: (i, 0)),
        ],
        out_shape=[
            jax.ShapeDtypeStruct((N, HID), jnp.float32),
            jax.ShapeDtypeStruct((N, XP), jnp.float32),
            jax.ShapeDtypeStruct((N, HP), jnp.float32),
            jax.ShapeDtypeStruct((N, HP), jnp.float32),
        ],
    )(op, pm, bias, hres, g, b, w, a_s, a_d)


def _ta_body(al8_ref, a_ref):
    a_ref[...] = jnp.sum(al8_ref[...], axis=1, keepdims=True)


def _ta_call(al8):
    BE = 20000
    return pl.pallas_call(
        _ta_body,
        grid=(E // BE,),
        in_specs=[pl.BlockSpec((BE, HP), lambda i: (i, 0))],
        out_specs=[pl.BlockSpec((BE, 1), lambda i: (i, 0))],
        out_shape=[jax.ShapeDtypeStruct((E, 1), jnp.float32)],
    )(al8)[0].reshape(E)


def _tf_body(op_ref, pm_ref, bias_ref, hres_ref, g_ref, b_ref, h_ref):
    agg = jnp.dot(op_ref[0] + op_ref[1], pm_ref[...],
                  preferred_element_type=jnp.float32)
    out = agg + bias_ref[...] + hres_ref[...]
    h_ref[...] = _ln(out, g_ref[...], b_ref[...])


def _tf_call(op, pm, bias, hres, g, b):
    BN = 2000
    return pl.pallas_call(
        _tf_body,
        grid=(N // BN,),
        in_specs=[
            pl.BlockSpec((2, BN, HID), lambda i: (0, i, 0)),
            pl.BlockSpec((HID, HID), lambda i: (0, 0)),
            pl.BlockSpec((1, HID), lambda i: (0, 0)),
            pl.BlockSpec((BN, HID), lambda i: (i, 0)),
            pl.BlockSpec((1, HID), lambda i: (0, 0)),
            pl.BlockSpec((1, HID), lambda i: (0, 0)),
        ],
        out_specs=[pl.BlockSpec((BN, HID), lambda i: (i, 0))],
        out_shape=[jax.ShapeDtypeStruct((N, HID), jnp.float32)],
    )(op, pm, bias, hres, g, b)[0]


def _s1_body(src_hbm, dst_hbm, es_hbm, ed_hbm,
             ex_hbm, sp_hbm,
             srcv_a, srcv_b, dstv_a, dstv_b, ga_a, ga_b, gb_a, gb_b,
             exv, zv, sacc, sem_sd, sem_g, sem_w):
    cid = lax.axis_index("c")
    sid = lax.axis_index("s")
    wid = sid * NC + cid

    def zrow(i, _):
        zv[i, :] = jnp.zeros((16,), jnp.float32)
        return 0
    lax.fori_loop(0, RPT, zrow, 0)
    pltpu.sync_copy(zv, sacc.at[pl.ds(sid * RPT, RPT)])
    plsc.subcore_barrier()

    base0 = wid * EPW

    pltpu.sync_copy(src_hbm.at[pl.ds(base0, C1)], srcv_a)
    pltpu.sync_copy(dst_hbm.at[pl.ds(base0, C1)], dstv_a)
    pltpu.async_copy(es_hbm.at[srcv_a], ga_a, sem_g)
    pltpu.async_copy(ed_hbm.at[dstv_a], gb_a, sem_g)

    def work(k, sv, dv, gav, gbv, svn, dvn, gan, gbn):
        b = base0 + k * C1
        more = k + 1 < NCH
        bn = b + C1

        @pl.when(more)
        def _():
            pltpu.async_copy(src_hbm.at[pl.ds(bn, C1)], svn, sem_sd)
            pltpu.async_copy(dst_hbm.at[pl.ds(bn, C1)], dvn, sem_sd)
        pltpu.make_async_copy(es_hbm.at[sv], gav, sem_g).wait()
        pltpu.make_async_copy(ed_hbm.at[dv], gbv, sem_g).wait()

        @pl.when(k >= 1)
        def _():
            pltpu.make_async_copy(
                exv, ex_hbm.at[pl.ds(base0, C1)], sem_w).wait()

        def erow(i, _):
            e = gav[i, :] + gbv[i, :]
            e = jnp.where(e >= 0.0, e, NEG * e)
            exv[i, :] = jnp.exp(e)
            return 0
        lax.fori_loop(0, C1, erow, 0)
        pltpu.async_copy(exv, ex_hbm.at[pl.ds(b, C1)], sem_w)
        pltpu.sync_copy(exv, sacc.at[dv], add=True)

        @pl.when(more)
        def _():
            pltpu.make_async_copy(src_hbm.at[pl.ds(bn, C1)], svn, sem_sd).wait()
            pltpu.make_async_copy(dst_hbm.at[pl.ds(bn, C1)], dvn, sem_sd).wait()
            pltpu.async_copy(es_hbm.at[svn], gan, sem_g)
            pltpu.async_copy(ed_hbm.at[dvn], gbn, sem_g)

    def chunk(k, _):
        q = k % 2

        @pl.when(q == 0)
        def _():
            work(k, srcv_a, dstv_a, ga_a, gb_a,
                 srcv_b, dstv_b, ga_b, gb_b)

        @pl.when(q == 1)
        def _():
            work(k, srcv_b, dstv_b, ga_b, gb_b,
                 srcv_a, dstv_a, ga_a, gb_a)
        return 0
    lax.fori_loop(0, NCH, chunk, 0)
    pltpu.make_async_copy(exv, ex_hbm.at[pl.ds(base0, C1)], sem_w).wait()
    plsc.subcore_barrier()

    @pl.when(sid == 0)
    def _():
        pltpu.sync_copy(sacc, sp_hbm.at[cid])


def _s1_call(src, dst, es, ed):
    k = pl.kernel(
        _s1_body,
        out_type=[
            jax.ShapeDtypeStruct((E, HP), jnp.float32),
            jax.ShapeDtypeStruct((NC, N, HP), jnp.float32),
        ],
        mesh=plsc.VectorSubcoreMesh(
            core_axis_name="c", subcore_axis_name="s",
            num_cores=NC, num_subcores=NS),
        compiler_params=pltpu.CompilerParams(use_tc_tiling_on_sc=False),
        scratch_types=[
            pltpu.VMEM((C1,), jnp.int32),
            pltpu.VMEM((C1,), jnp.int32),
            pltpu.VMEM((C1,), jnp.int32),
            pltpu.VMEM((C1,), jnp.int32),
            pltpu.VMEM((C1, HP), jnp.float32),
            pltpu.VMEM((C1, HP), jnp.float32),
            pltpu.VMEM((C1, HP), jnp.float32),
            pltpu.VMEM((C1, HP), jnp.float32),
            pltpu.VMEM((C1, HP), jnp.float32),
            pltpu.VMEM((RPT, HP), jnp.float32),
            pltpu.VMEM_SHARED((N, HP), jnp.float32),
            pltpu.SemaphoreType.DMA,
            pltpu.SemaphoreType.DMA,
            pltpu.SemaphoreType.DMA,
        ],
    )
    return k(src, dst, es, ed)


def _s2_body(src_hbm, dst_hbm, ex_hbm, r8_hbm, xp_hbm, outs, scratch):
    op_hbm = outs[0]
    al_hbm = outs[1] if len(outs) > 1 else None
    (srcv_a, srcv_b, dstv_a, dstv_b, exv, rv, av, xg0, xg1, msg0, msg1,
     zv, oacc, gs0, gs1, ss0, ss1, sem_e, sem_r, sem_sd) = scratch
    xg = [xg0, xg1]
    msg = [msg0, msg1]
    gsem = [gs0, gs1]
    ssem = [ss0, ss1]
    cid = lax.axis_index("c")
    sid = lax.axis_index("s")
    wid = sid * NC + cid

    def zrow(i, _):
        for j in range(HID // 16):
            zv[i, pl.ds(j * 16, 16)] = jnp.zeros((16,), jnp.float32)
        return 0
    lax.fori_loop(0, 25, zrow, 0)

    def zcp(r, _):
        pltpu.sync_copy(zv, oacc.at[pl.ds(sid * RPT + r * 25, 25)])
        return 0
    lax.fori_loop(0, RPT // 25, zcp, 0)
    plsc.subcore_barrier()

    base0 = wid * EPW

    pltpu.sync_copy(src_hbm.at[pl.ds(base0, C1)], srcv_a)
    pltpu.sync_copy(dst_hbm.at[pl.ds(base0, C1)], dstv_a)
    pltpu.async_copy(ex_hbm.at[pl.ds(base0, C1)], exv, sem_e)
    pltpu.async_copy(r8_hbm.at[dstv_a], rv, sem_r)

    def subblock(k, sv, dv, svn, dvn):
        more = k + 1 < NCH
        bn = base0 + (k + 1) * C1
        gd = [None, None]
        sd = [None, None]
        gd[0] = pltpu.async_copy(
            xp_hbm.at[sv.at[pl.ds(0, SUB)]], xg[0], gsem[0])
        for s in range(NSUB):
            p = s % 2
            if s + 1 < NSUB:
                gd[1 - p] = pltpu.async_copy(
                    xp_hbm.at[sv.at[pl.ds((s + 1) * SUB, SUB)]],
                    xg[1 - p], gsem[1 - p])
            if s == 0:
                @pl.when(more)
                def _():
                    pltpu.async_copy(src_hbm.at[pl.ds(bn, C1)], svn, sem_sd)
                    pltpu.async_copy(dst_hbm.at[pl.ds(bn, C1)], dvn, sem_sd)
            if s == 3:
                @pl.when(more)
                def _():
                    pltpu.make_async_copy(
                        src_hbm.at[pl.ds(bn, C1)], svn, sem_sd).wait()
                    pltpu.make_async_copy(
                        dst_hbm.at[pl.ds(bn, C1)], dvn, sem_sd).wait()
                    pltpu.async_copy(ex_hbm.at[pl.ds(bn, C1)], exv, sem_e)
                    pltpu.async_copy(r8_hbm.at[dvn], rv, sem_r)
            gd[p].wait()
            if s >= 2:
                sd[p].wait()
            xgp = xg[p]
            msgp = msg[p]

            def edge(e2, _):
                arow = av[s * SUB + e2, :]
                for j in range(HID // 16):
                    acc = arow[0] * xgp[e2, pl.ds(j * 16, 16)]
                    for h in range(1, HEADS):
                        acc = acc + arow[h] * xgp[e2, pl.ds(h * HID + j * 16, 16)]
                    msgp[e2, pl.ds(j * 16, 16)] = acc
                return 0
            lax.fori_loop(0, SUB, edge, 0)
            sd[p] = pltpu.async_copy(
                msgp, oacc.at[dv[pl.ds(s * SUB, SUB)]], ssem[p], add=True)
        sd[(NSUB - 2) % 2].wait()
        sd[(NSUB - 1) % 2].wait()

    def chunk(k, _):
        q = k % 2
        b = base0 + k * C1
        pltpu.make_async_copy(ex_hbm.at[pl.ds(b, C1)], exv, sem_e).wait()

        @pl.when(q == 0)
        def _():
            pltpu.make_async_copy(r8_hbm.at[dstv_a], rv, sem_r).wait()

        @pl.when(q == 1)
        def _():
            pltpu.make_async_copy(r8_hbm.at[dstv_b], rv, sem_r).wait()

        def arow(i, _):
            av[i, :] = exv[i, :] * rv[i, :]
            return 0
        lax.fori_loop(0, C1, arow, 0)
        if al_hbm is not None:
            pltpu.sync_copy(av, al_hbm.at[pl.ds(b, C1)])

        @pl.when(q == 0)
        def _():
            subblock(k, srcv_a, dstv_a, srcv_b, dstv_b)

        @pl.when(q == 1)
        def _():
            subblock(k, srcv_b, dstv_b, srcv_a, dstv_a)
        return 0
    lax.fori_loop(0, NCH, chunk, 0)
    plsc.subcore_barrier()

    @pl.when(sid == 0)
    def _():
        pltpu.sync_copy(oacc, op_hbm.at[cid])


def _s2_call(src, dst, ex, r8, xp, want_alpha):
    out_type = [jax.ShapeDtypeStruct((NC, N, HID), jnp.float32)]
    if want_alpha:
        out_type.append(jax.ShapeDtypeStruct((E, HP), jnp.float32))

    def body(*refs):
        ins = refs[:5]
        outs = refs[5:5 + len(out_type)]
        scratch = refs[5 + len(out_type):]
        _s2_body(*ins, outs, scratch)

    k = pl.kernel(
        body,
        out_type=out_type,
        mesh=plsc.VectorSubcoreMesh(
            core_axis_name="c", subcore_axis_name="s",
            num_cores=NC, num_subcores=NS),
        compiler_params=pltpu.CompilerParams(use_tc_tiling_on_sc=False),
        scratch_types=[
            pltpu.VMEM((C1,), jnp.int32),
            pltpu.VMEM((C1,), jnp.int32),
            pltpu.VMEM((C1,), jnp.int32),
            pltpu.VMEM((C1,), jnp.int32),
            pltpu.VMEM((C1, HP), jnp.float32),
            pltpu.VMEM((C1, HP), jnp.float32),
            pltpu.VMEM((C1, HP), jnp.float32),
            pltpu.VMEM((SUB, XP), jnp.float32),
            pltpu.VMEM((SUB, XP), jnp.float32),
            pltpu.VMEM((SUB, HID), jnp.float32),
            pltpu.VMEM((SUB, HID), jnp.float32),
            pltpu.VMEM((25, HID), jnp.float32),
            pltpu.VMEM_SHARED((N, HID), jnp.float32),
            pltpu.SemaphoreType.DMA,
            pltpu.SemaphoreType.DMA,
            pltpu.SemaphoreType.DMA,
            pltpu.SemaphoreType.DMA,
            pltpu.SemaphoreType.DMA,
            pltpu.SemaphoreType.DMA,
            pltpu.SemaphoreType.DMA,
        ],
    )
    return k(src, dst, ex, r8, xp)


def _unperm_mat():
    return jnp.eye(HID, dtype=jnp.float32)


def _att_mats(a_src, a_dst):
    eye = jnp.eye(HEADS, dtype=jnp.float32)
    asm = jnp.einsum('hc,hk->hck', a_src, eye).reshape(XP, HEADS)
    adm = jnp.einsum('hc,hk->hck', a_dst, eye).reshape(XP, HEADS)
    pad = jnp.zeros((XP, HP - HEADS), jnp.float32)
    return (jnp.concatenate([asm, pad], axis=1),
            jnp.concatenate([adm, pad], axis=1))


def kernel(x, edge_index, proj_W, proj_b, ln_in_g, ln_in_b,
           W0, att_src0, att_dst0, bias0, ln0_g, ln0_b,
           W1, att_src1, att_dst1, bias1, ln1_g, ln1_b,
           W2, att_src2, att_dst2, bias2, ln2_g, ln2_b):
    src = edge_index[0]
    dst = edge_index[1]
    Ws = [W0, W1, W2]
    atts = [(att_src0, att_dst0), (att_src1, att_dst1), (att_src2, att_dst2)]
    biases = [bias0, bias1, bias2]
    lns = [(ln0_g, ln0_b), (ln1_g, ln1_b), (ln2_g, ln2_b)]

    pm = _unperm_mat()
    a_s0, a_d0 = _att_mats(*atts[0])
    h, xp, es, ed = _t0_call(
        x, proj_W, proj_b.reshape(1, HID),
        ln_in_g.reshape(1, HID), ln_in_b.reshape(1, HID), W0, a_s0, a_d0)

    alpha = None
    for i in range(3):
        ex, s_part = _s1_call(src, dst, es, ed)
        r8 = _t2_call(s_part)
        if i < 2:
            op = _s2_call(src, dst, ex, r8, xp, False)[0]
            a_s, a_d = _att_mats(*atts[i + 1])
            h, xp, es, ed = _t3_call(
                op, pm, biases[i].reshape(1, HID), h,
                lns[i][0].reshape(1, HID), lns[i][1].reshape(1, HID),
                Ws[i + 1], a_s, a_d)
        else:
            op, al8 = _s2_call(src, dst, ex, r8, xp, True)
            alpha = _ta_call(al8)
            h = _tf_call(op, pm, biases[i].reshape(1, HID), h,
                         lns[i][0].reshape(1, HID), lns[i][1].reshape(1, HID))
    return h, alpha
